# baseline (speedup 1.0000x reference)
import functools

import jax
import jax.numpy as jnp
from jax import lax
from jax.experimental import pallas as pl
from jax.experimental.pallas import tpu as pltpu
from jax.experimental.pallas import tpu_sc as plsc

T = 2048
H = 2048
F = 512
E = 8
K = 2
LANES = 128
BT = 256
BR = 128
S_PAD = T * K + E * BR
NB = S_PAD // BR
NT = T // BT

_SC_CORES = 2
_SC_SUBCORES = 16
NW = _SC_CORES * _SC_SUBCORES
TPW = T // NW
NCD = 16
NCHD = TPW // NCD
NCC = 8
NCHC = TPW // NCC


def _count_kernel(x_ref, gate_ref, probs_ref, cnt_ref):
    t = pl.program_id(0)
    lane = lax.broadcasted_iota(jnp.int32, (BT, LANES), 1)
    valid = lane < E
    xv = x_ref[...]
    logits = jnp.dot(xv, gate_ref[...], preferred_element_type=jnp.float32)
    logits = jnp.where(valid, logits, -1e30)
    m1 = jnp.max(logits, axis=1, keepdims=True)
    p = jnp.where(valid, jnp.exp(logits - m1), 0.0)
    probs = p / jnp.sum(p, axis=1, keepdims=True)
    probs_ref[...] = probs

    v1 = jnp.max(probs, axis=1, keepdims=True)
    i1 = jnp.min(jnp.where(probs == v1, lane, LANES), axis=1, keepdims=True)
    probs2 = jnp.where(lane == i1, -1.0, probs)
    v2 = jnp.max(probs2, axis=1, keepdims=True)
    i2 = jnp.min(jnp.where(probs2 == v2, lane, LANES), axis=1, keepdims=True)
    onehot = ((lane == i1) | (lane == i2)).astype(jnp.float32)

    blk = jnp.sum(onehot, axis=0, keepdims=True)
    blk8 = jnp.broadcast_to(blk, (8, LANES))
    row = lax.broadcasted_iota(jnp.int32, (8, LANES), 0)
    blk8 = jnp.where(row == 0, blk8, 0.0)

    @pl.when(t == 0)
    def _init():
        cnt_ref[...] = blk8

    @pl.when(t != 0)
    def _acc():
        cnt_ref[...] += blk8


def _count_call(x, gate_pad):
    return pl.pallas_call(
        _count_kernel,
        grid=(NT,),
        in_specs=[
            pl.BlockSpec((BT, H), lambda t: (t, 0)),
            pl.BlockSpec((H, LANES), lambda t: (0, 0)),
        ],
        out_specs=[
            pl.BlockSpec((BT, LANES), lambda t: (t, 0)),
            pl.BlockSpec((8, LANES), lambda t: (0, 0)),
        ],
        out_shape=[
            jax.ShapeDtypeStruct((T, LANES), jnp.float32),
            jax.ShapeDtypeStruct((8, LANES), jnp.float32),
        ],
    )(x, gate_pad)


def _emit_kernel(probs_ref, cnt_ref, idx_ref, w0_ref, w1_ref, s_run):
    t = pl.program_id(0)
    lane = lax.broadcasted_iota(jnp.int32, (BT, LANES), 1)
    probs = probs_ref[...]
    v1 = jnp.max(probs, axis=1, keepdims=True)
    i1 = jnp.min(jnp.where(probs == v1, lane, LANES), axis=1, keepdims=True)
    probs2 = jnp.where(lane == i1, -1.0, probs)
    v2 = jnp.max(probs2, axis=1, keepdims=True)
    i2 = jnp.min(jnp.where(probs2 == v2, lane, LANES), axis=1, keepdims=True)
    onehot = ((lane == i1) | (lane == i2)).astype(jnp.float32)

    @pl.when(t == 0)
    def _offsets():
        counts = cnt_ref[0:1, :]
        padded = jnp.floor((counts + (BR - 1)) * (1.0 / BR)) * BR
        r = lax.broadcasted_iota(jnp.int32, (LANES, LANES), 0)
        c = lax.broadcasted_iota(jnp.int32, (LANES, LANES), 1)
        upper = (r < c).astype(jnp.float32)
        padded8 = jnp.broadcast_to(padded, (8, LANES))
        offs8 = jnp.dot(padded8, upper, preferred_element_type=jnp.float32)
        s_run[...] = offs8[0:1, :]

    base = s_run[...]
    rb = lax.broadcasted_iota(jnp.int32, (BT, BT), 0)
    cb = lax.broadcasted_iota(jnp.int32, (BT, BT), 1)
    lower = (cb < rb).astype(jnp.float32)
    rank = jnp.dot(lower, onehot, preferred_element_type=jnp.float32)
    pos = base + rank
    d0 = jnp.sum(jnp.where(lane == i1, pos, 0.0), axis=1, keepdims=True)
    d1 = jnp.sum(jnp.where(lane == i2, pos, 0.0), axis=1, keepdims=True)
    denom = v1 + v2 + 1e-20
    idx_ref[...] = (jnp.where(lane == 0, d0, 0.0)
                    + jnp.where(lane == 1, d1, 0.0)).astype(jnp.int32)
    w0_ref[...] = jnp.broadcast_to(v1 / denom, (BT, 16))
    w1_ref[...] = jnp.broadcast_to(v2 / denom, (BT, 16))
    s_run[...] += jnp.sum(onehot, axis=0, keepdims=True)


def _emit_call(probs, cnt):
    return pl.pallas_call(
        _emit_kernel,
        grid=(NT,),
        in_specs=[
            pl.BlockSpec((BT, LANES), lambda t: (t, 0)),
            pl.BlockSpec((8, LANES), lambda t: (0, 0)),
        ],
        out_specs=[
            pl.BlockSpec((BT, LANES), lambda t: (t, 0)),
            pl.BlockSpec((BT, 16), lambda t: (t, 0)),
            pl.BlockSpec((BT, 16), lambda t: (t, 0)),
        ],
        out_shape=[
            jax.ShapeDtypeStruct((T, LANES), jnp.int32),
            jax.ShapeDtypeStruct((T, 16), jnp.float32),
            jax.ShapeDtypeStruct((T, 16), jnp.float32),
        ],
        scratch_shapes=[pltpu.VMEM((1, LANES), jnp.float32)],
    )(probs, cnt)


def _dispatch_call(x, d0, d1):
    mesh = plsc.VectorSubcoreMesh(core_axis_name="c", subcore_axis_name="s")

    @functools.partial(
        pl.kernel,
        mesh=mesh,
        out_type=jax.ShapeDtypeStruct((S_PAD, H), jnp.float32),
        scratch_types=[
            pltpu.VMEM((NCHD, NCD), jnp.int32),
            pltpu.VMEM((NCHD, NCD), jnp.int32),
            pltpu.VMEM((NCD, H), jnp.float32),
            pltpu.VMEM((NCD, H), jnp.float32),
            pltpu.SemaphoreType.DMA,
            pltpu.SemaphoreType.DMA,
            pltpu.SemaphoreType.DMA,
            pltpu.SemaphoreType.DMA,
            pltpu.SemaphoreType.DMA,
            pltpu.SemaphoreType.DMA,
        ],
    )
    def dispatch(x_hbm, d0_hbm, d1_hbm, xs_hbm,
                 idx0_v, idx1_v, rows_a, rows_b,
                 ld_a, ld_b, s0_a, s0_b, s1_a, s1_b):
        wid = lax.axis_index("s") * _SC_CORES + lax.axis_index("c")
        base = wid * TPW
        pltpu.sync_copy(d0_hbm.at[wid], idx0_v)
        pltpu.sync_copy(d1_hbm.at[wid], idx1_v)
        rows = [rows_a, rows_b]
        ld = [ld_a, ld_b]
        s0 = [s0_a, s0_b]
        s1 = [s1_a, s1_b]

        def issue_load(j, s):
            return pltpu.async_copy(
                x_hbm.at[pl.ds(base + j * NCD, NCD)], rows[s], ld[s])

        load_h = {0: issue_load(0, 0)}
        scat_h = {}
        for j in range(NCHD):
            s = j & 1
            if j + 1 < NCHD:
                if j >= 1:
                    for hh in scat_h.pop(j - 1):
                        hh.wait()
                load_h[j + 1] = issue_load(j + 1, 1 - s)
            load_h.pop(j).wait()
            scat_h[j] = [
                pltpu.async_copy(rows[s], xs_hbm.at[idx0_v.at[j]], s0[s]),
                pltpu.async_copy(rows[s], xs_hbm.at[idx1_v.at[j]], s1[s]),
            ]
        for hs in scat_h.values():
            for hh in hs:
                hh.wait()

    return dispatch(x, d0, d1)


def _gmm_kernel(be_ref, xs_ref, wg_ref, wu_ref, wd_ref, ys_ref):
    del be_ref
    xv = xs_ref[...]
    g = jnp.dot(xv, wg_ref[0], preferred_element_type=jnp.float32)
    u = jnp.dot(xv, wu_ref[0], preferred_element_type=jnp.float32)
    h = g * (1.0 / (1.0 + jnp.exp(-g))) * u
    ys_ref[...] = jnp.dot(h, wd_ref[0], preferred_element_type=jnp.float32)


def _gmm_call(be, xs, W_gate, W_up, W_down):
    grid_spec = pltpu.PrefetchScalarGridSpec(
        num_scalar_prefetch=1,
        grid=(NB,),
        in_specs=[
            pl.BlockSpec((BR, H), lambda b, be: (b, 0)),
            pl.BlockSpec((1, H, F), lambda b, be: (be[b], 0, 0)),
            pl.BlockSpec((1, H, F), lambda b, be: (be[b], 0, 0)),
            pl.BlockSpec((1, F, H), lambda b, be: (be[b], 0, 0)),
        ],
        out_specs=pl.BlockSpec((BR, H), lambda b, be: (b, 0)),
    )
    return pl.pallas_call(
        _gmm_kernel,
        grid_spec=grid_spec,
        out_shape=jax.ShapeDtypeStruct((S_PAD, H), jnp.float32),
    )(be, xs, W_gate, W_up, W_down)


def _combine_call(ys, d0, d1, w0r, w1r):
    mesh = plsc.VectorSubcoreMesh(core_axis_name="c", subcore_axis_name="s")

    @functools.partial(
        pl.kernel,
        mesh=mesh,
        out_type=jax.ShapeDtypeStruct((T, H), jnp.float32),
        scratch_types=[
            pltpu.VMEM((NCHC, NCC), jnp.int32),
            pltpu.VMEM((NCHC, NCC), jnp.int32),
            pltpu.VMEM((TPW, 16), jnp.float32),
            pltpu.VMEM((TPW, 16), jnp.float32),
            pltpu.VMEM((NCC, H), jnp.float32),
            pltpu.VMEM((NCC, H), jnp.float32),
            pltpu.VMEM((NCC, H), jnp.float32),
            pltpu.VMEM((NCC, H), jnp.float32),
            pltpu.VMEM((NCC, H), jnp.float32),
            pltpu.VMEM((NCC, H), jnp.float32),
            pltpu.SemaphoreType.DMA,
            pltpu.SemaphoreType.DMA,
            pltpu.SemaphoreType.DMA,
            pltpu.SemaphoreType.DMA,
            pltpu.SemaphoreType.DMA,
            pltpu.SemaphoreType.DMA,
        ],
    )
    def combine(ys_hbm, d0_hbm, d1_hbm, w0_hbm, w1_hbm, out_hbm,
                idx0_v, idx1_v, w0_v, w1_v,
                b0_a, b0_b, b1_a, b1_b, ob_a, ob_b,
                g0_a, g0_b, g1_a, g1_b, st_a, st_b):
        wid = lax.axis_index("s") * _SC_CORES + lax.axis_index("c")
        base = wid * TPW
        pltpu.sync_copy(d0_hbm.at[wid], idx0_v)
        pltpu.sync_copy(d1_hbm.at[wid], idx1_v)
        pltpu.sync_copy(w0_hbm.at[wid], w0_v)
        pltpu.sync_copy(w1_hbm.at[wid], w1_v)
        b0 = [b0_a, b0_b]
        b1 = [b1_a, b1_b]
        ob = [ob_a, ob_b]
        g0 = [g0_a, g0_b]
        g1 = [g1_a, g1_b]
        st = [st_a, st_b]

        def issue_gather(j, s):
            return [
                pltpu.async_copy(ys_hbm.at[idx0_v.at[j]], b0[s], g0[s]),
                pltpu.async_copy(ys_hbm.at[idx1_v.at[j]], b1[s], g1[s]),
            ]

        gath_h = {0: issue_gather(0, 0)}
        store_h = {}
        for j in range(NCHC):
            s = j & 1
            if j + 1 < NCHC:
                gath_h[j + 1] = issue_gather(j + 1, 1 - s)
            for hh in gath_h.pop(j):
                hh.wait()
            if j >= 2:
                store_h.pop(j - 2).wait()
            for i in range(NCC):
                w0s = w0_v[j * NCC + i]
                w1s = w1_v[j * NCC + i]

                def body(q, _, s=s, i=i, w0s=w0s, w1s=w1s):
                    a = b0[s][i, pl.ds(q * 16, 16)]
                    b = b1[s][i, pl.ds(q * 16, 16)]
                    ob[s][i, pl.ds(q * 16, 16)] = w0s * a + w1s * b
                    return 0

                lax.fori_loop(0, H // 16, body, 0, unroll=1)
            store_h[j] = pltpu.async_copy(
                ob[s], out_hbm.at[pl.ds(base + j * NCC, NCC)], st[s])
        for hh in store_h.values():
            hh.wait()

    return combine(ys, d0, d1, w0r, w1r)


@jax.jit
def kernel(x, gate_tensor, W_gate, W_up, W_down):
    gate_pad = jnp.zeros((H, LANES), jnp.float32).at[:, :E].set(gate_tensor)
    probs, cnt = _count_call(x, gate_pad)
    idx_out, w0b, w1b = _emit_call(probs, cnt)

    counts = cnt[0, :E].astype(jnp.int32)
    padded = ((counts + BR - 1) // BR) * BR
    ends = jnp.cumsum(padded)
    starts = jnp.arange(NB, dtype=jnp.int32) * BR
    be = jnp.sum((ends[None, :] <= starts[:, None]).astype(jnp.int32), axis=1)
    be = jnp.minimum(be, E - 1).astype(jnp.int32)

    d0 = idx_out[:, 0].reshape(NW, NCHD, NCD)
    d1 = idx_out[:, 1].reshape(NW, NCHD, NCD)
    d0c = idx_out[:, 0].reshape(NW, NCHC, NCC)
    d1c = idx_out[:, 1].reshape(NW, NCHC, NCC)
    w0r = w0b.reshape(NW, TPW, 16)
    w1r = w1b.reshape(NW, TPW, 16)

    xs = _dispatch_call(x, d0, d1)
    ys = _gmm_call(be, xs, W_gate, W_up, W_down)
    return _combine_call(ys, d0c, d1c, w0r, w1r)


# speedup vs baseline: 1.0261x; 1.0261x over previous
import functools

import jax
import jax.numpy as jnp
from jax import lax
from jax.experimental import pallas as pl
from jax.experimental.pallas import tpu as pltpu
from jax.experimental.pallas import tpu_sc as plsc

T = 2048
H = 2048
F = 512
E = 8
K = 2
LANES = 128
BT = 256
BR = 128
S_PAD = T * K + E * BR
NB = S_PAD // BR
NT = T // BT

_SC_CORES = 2
_SC_SUBCORES = 16
NW = _SC_CORES * _SC_SUBCORES
TPW = T // NW
NCD = 8
NCHD = TPW // NCD
NCC = 8
NCHC = TPW // NCC


def _count_kernel(x_ref, gate_ref, probs_ref, cnt_ref):
    t = pl.program_id(0)
    lane = lax.broadcasted_iota(jnp.int32, (BT, LANES), 1)
    valid = lane < E
    xv = x_ref[...]
    logits = jnp.dot(xv, gate_ref[...], preferred_element_type=jnp.float32)
    logits = jnp.where(valid, logits, -1e30)
    m1 = jnp.max(logits, axis=1, keepdims=True)
    p = jnp.where(valid, jnp.exp(logits - m1), 0.0)
    probs = p / jnp.sum(p, axis=1, keepdims=True)
    probs_ref[...] = probs

    v1 = jnp.max(probs, axis=1, keepdims=True)
    i1 = jnp.min(jnp.where(probs == v1, lane, LANES), axis=1, keepdims=True)
    probs2 = jnp.where(lane == i1, -1.0, probs)
    v2 = jnp.max(probs2, axis=1, keepdims=True)
    i2 = jnp.min(jnp.where(probs2 == v2, lane, LANES), axis=1, keepdims=True)
    onehot = ((lane == i1) | (lane == i2)).astype(jnp.float32)

    blk = jnp.sum(onehot, axis=0, keepdims=True)
    blk8 = jnp.broadcast_to(blk, (8, LANES))
    row = lax.broadcasted_iota(jnp.int32, (8, LANES), 0)
    blk8 = jnp.where(row == 0, blk8, 0.0)

    @pl.when(t == 0)
    def _init():
        cnt_ref[...] = blk8

    @pl.when(t != 0)
    def _acc():
        cnt_ref[...] += blk8


def _count_call(x, gate_pad):
    return pl.pallas_call(
        _count_kernel,
        grid=(NT,),
        in_specs=[
            pl.BlockSpec((BT, H), lambda t: (t, 0)),
            pl.BlockSpec((H, LANES), lambda t: (0, 0)),
        ],
        out_specs=[
            pl.BlockSpec((BT, LANES), lambda t: (t, 0)),
            pl.BlockSpec((8, LANES), lambda t: (0, 0)),
        ],
        out_shape=[
            jax.ShapeDtypeStruct((T, LANES), jnp.float32),
            jax.ShapeDtypeStruct((8, LANES), jnp.float32),
        ],
    )(x, gate_pad)


def _emit_kernel(probs_ref, cnt_ref, idx_ref, w0_ref, w1_ref, s_run):
    t = pl.program_id(0)
    lane = lax.broadcasted_iota(jnp.int32, (BT, LANES), 1)
    probs = probs_ref[...]
    v1 = jnp.max(probs, axis=1, keepdims=True)
    i1 = jnp.min(jnp.where(probs == v1, lane, LANES), axis=1, keepdims=True)
    probs2 = jnp.where(lane == i1, -1.0, probs)
    v2 = jnp.max(probs2, axis=1, keepdims=True)
    i2 = jnp.min(jnp.where(probs2 == v2, lane, LANES), axis=1, keepdims=True)
    onehot = ((lane == i1) | (lane == i2)).astype(jnp.float32)

    @pl.when(t == 0)
    def _offsets():
        counts = cnt_ref[0:1, :]
        padded = jnp.floor((counts + (BR - 1)) * (1.0 / BR)) * BR
        r = lax.broadcasted_iota(jnp.int32, (LANES, LANES), 0)
        c = lax.broadcasted_iota(jnp.int32, (LANES, LANES), 1)
        upper = (r < c).astype(jnp.float32)
        padded8 = jnp.broadcast_to(padded, (8, LANES))
        offs8 = jnp.dot(padded8, upper, preferred_element_type=jnp.float32)
        s_run[...] = offs8[0:1, :]

    base = s_run[...]
    rb = lax.broadcasted_iota(jnp.int32, (BT, BT), 0)
    cb = lax.broadcasted_iota(jnp.int32, (BT, BT), 1)
    lower = (cb < rb).astype(jnp.float32)
    rank = jnp.dot(lower, onehot, preferred_element_type=jnp.float32)
    pos = base + rank
    d0 = jnp.sum(jnp.where(lane == i1, pos, 0.0), axis=1, keepdims=True)
    d1 = jnp.sum(jnp.where(lane == i2, pos, 0.0), axis=1, keepdims=True)
    denom = v1 + v2 + 1e-20
    idx_ref[...] = (jnp.where(lane == 0, d0, 0.0)
                    + jnp.where(lane == 1, d1, 0.0)).astype(jnp.int32)
    w0_ref[...] = jnp.broadcast_to(v1 / denom, (BT, 16))
    w1_ref[...] = jnp.broadcast_to(v2 / denom, (BT, 16))
    s_run[...] += jnp.sum(onehot, axis=0, keepdims=True)


def _emit_call(probs, cnt):
    return pl.pallas_call(
        _emit_kernel,
        grid=(NT,),
        in_specs=[
            pl.BlockSpec((BT, LANES), lambda t: (t, 0)),
            pl.BlockSpec((8, LANES), lambda t: (0, 0)),
        ],
        out_specs=[
            pl.BlockSpec((BT, LANES), lambda t: (t, 0)),
            pl.BlockSpec((BT, 16), lambda t: (t, 0)),
            pl.BlockSpec((BT, 16), lambda t: (t, 0)),
        ],
        out_shape=[
            jax.ShapeDtypeStruct((T, LANES), jnp.int32),
            jax.ShapeDtypeStruct((T, 16), jnp.float32),
            jax.ShapeDtypeStruct((T, 16), jnp.float32),
        ],
        scratch_shapes=[pltpu.VMEM((1, LANES), jnp.float32)],
    )(probs, cnt)


def _dispatch_call(x, d0, d1):
    mesh = plsc.VectorSubcoreMesh(core_axis_name="c", subcore_axis_name="s")

    @functools.partial(
        pl.kernel,
        mesh=mesh,
        out_type=jax.ShapeDtypeStruct((S_PAD, H), jnp.float32),
        scratch_types=[
            pltpu.VMEM((NCHD, NCD), jnp.int32),
            pltpu.VMEM((NCHD, NCD), jnp.int32),
            pltpu.VMEM((NCD, H), jnp.float32),
            pltpu.VMEM((NCD, H), jnp.float32),
            pltpu.SemaphoreType.DMA,
            pltpu.SemaphoreType.DMA,
            pltpu.SemaphoreType.DMA,
            pltpu.SemaphoreType.DMA,
            pltpu.SemaphoreType.DMA,
            pltpu.SemaphoreType.DMA,
        ],
    )
    def dispatch(x_hbm, d0_hbm, d1_hbm, xs_hbm,
                 idx0_v, idx1_v, rows_a, rows_b,
                 ld_a, ld_b, s0_a, s0_b, s1_a, s1_b):
        wid = lax.axis_index("s") * _SC_CORES + lax.axis_index("c")
        base = wid * TPW
        pltpu.sync_copy(d0_hbm.at[wid], idx0_v)
        pltpu.sync_copy(d1_hbm.at[wid], idx1_v)
        rows = [rows_a, rows_b]
        ld = [ld_a, ld_b]
        s0 = [s0_a, s0_b]
        s1 = [s1_a, s1_b]

        def issue_load(j, s):
            return pltpu.async_copy(
                x_hbm.at[pl.ds(base + j * NCD, NCD)], rows[s], ld[s])

        load_h = {0: issue_load(0, 0)}
        scat_h = {}
        for j in range(NCHD):
            s = j & 1
            if j + 1 < NCHD:
                if j >= 1:
                    for hh in scat_h.pop(j - 1):
                        hh.wait()
                load_h[j + 1] = issue_load(j + 1, 1 - s)
            load_h.pop(j).wait()
            scat_h[j] = [
                pltpu.async_copy(rows[s], xs_hbm.at[idx0_v.at[j]], s0[s]),
                pltpu.async_copy(rows[s], xs_hbm.at[idx1_v.at[j]], s1[s]),
            ]
        for hs in scat_h.values():
            for hh in hs:
                hh.wait()

    return dispatch(x, d0, d1)


def _gmm_kernel(be_ref, xs_ref, wg_ref, wu_ref, wd_ref, ys_ref):
    del be_ref
    xv = xs_ref[...]
    g = jnp.dot(xv, wg_ref[0], preferred_element_type=jnp.float32)
    u = jnp.dot(xv, wu_ref[0], preferred_element_type=jnp.float32)
    h = g * (1.0 / (1.0 + jnp.exp(-g))) * u
    ys_ref[...] = jnp.dot(h, wd_ref[0], preferred_element_type=jnp.float32)


def _gmm_call(be, xs, W_gate, W_up, W_down):
    grid_spec = pltpu.PrefetchScalarGridSpec(
        num_scalar_prefetch=1,
        grid=(NB,),
        in_specs=[
            pl.BlockSpec((BR, H), lambda b, be: (b, 0)),
            pl.BlockSpec((1, H, F), lambda b, be: (be[b], 0, 0)),
            pl.BlockSpec((1, H, F), lambda b, be: (be[b], 0, 0)),
            pl.BlockSpec((1, F, H), lambda b, be: (be[b], 0, 0)),
        ],
        out_specs=pl.BlockSpec((BR, H), lambda b, be: (b, 0)),
    )
    return pl.pallas_call(
        _gmm_kernel,
        grid_spec=grid_spec,
        out_shape=jax.ShapeDtypeStruct((S_PAD, H), jnp.float32),
    )(be, xs, W_gate, W_up, W_down)


def _combine_call(ys, d0, d1, w0r, w1r):
    mesh = plsc.VectorSubcoreMesh(core_axis_name="c", subcore_axis_name="s")

    @functools.partial(
        pl.kernel,
        mesh=mesh,
        out_type=jax.ShapeDtypeStruct((T, H), jnp.float32),
        scratch_types=[
            pltpu.VMEM((NCHC, NCC), jnp.int32),
            pltpu.VMEM((NCHC, NCC), jnp.int32),
            pltpu.VMEM((TPW, 16), jnp.float32),
            pltpu.VMEM((TPW, 16), jnp.float32),
            pltpu.VMEM((NCC, H), jnp.float32),
            pltpu.VMEM((NCC, H), jnp.float32),
            pltpu.VMEM((NCC, H), jnp.float32),
            pltpu.VMEM((NCC, H), jnp.float32),
            pltpu.VMEM((NCC, H), jnp.float32),
            pltpu.VMEM((NCC, H), jnp.float32),
            pltpu.SemaphoreType.DMA,
            pltpu.SemaphoreType.DMA,
            pltpu.SemaphoreType.DMA,
            pltpu.SemaphoreType.DMA,
            pltpu.SemaphoreType.DMA,
            pltpu.SemaphoreType.DMA,
        ],
    )
    def combine(ys_hbm, d0_hbm, d1_hbm, w0_hbm, w1_hbm, out_hbm,
                idx0_v, idx1_v, w0_v, w1_v,
                b0_a, b0_b, b1_a, b1_b, ob_a, ob_b,
                g0_a, g0_b, g1_a, g1_b, st_a, st_b):
        wid = lax.axis_index("s") * _SC_CORES + lax.axis_index("c")
        base = wid * TPW
        pltpu.sync_copy(d0_hbm.at[wid], idx0_v)
        pltpu.sync_copy(d1_hbm.at[wid], idx1_v)
        pltpu.sync_copy(w0_hbm.at[wid], w0_v)
        pltpu.sync_copy(w1_hbm.at[wid], w1_v)
        b0 = [b0_a, b0_b]
        b1 = [b1_a, b1_b]
        ob = [ob_a, ob_b]
        g0 = [g0_a, g0_b]
        g1 = [g1_a, g1_b]
        st = [st_a, st_b]

        def issue_gather(j, s):
            return [
                pltpu.async_copy(ys_hbm.at[idx0_v.at[j]], b0[s], g0[s]),
                pltpu.async_copy(ys_hbm.at[idx1_v.at[j]], b1[s], g1[s]),
            ]

        gath_h = {0: issue_gather(0, 0)}
        store_h = {}
        for j in range(NCHC):
            s = j & 1
            if j + 1 < NCHC:
                gath_h[j + 1] = issue_gather(j + 1, 1 - s)
            for hh in gath_h.pop(j):
                hh.wait()
            if j >= 2:
                store_h.pop(j - 2).wait()
            for i in range(NCC):
                w0s = w0_v[j * NCC + i]
                w1s = w1_v[j * NCC + i]

                def body(q, _, s=s, i=i, w0s=w0s, w1s=w1s):
                    a = b0[s][i, pl.ds(q * 16, 16)]
                    b = b1[s][i, pl.ds(q * 16, 16)]
                    ob[s][i, pl.ds(q * 16, 16)] = w0s * a + w1s * b
                    return 0

                lax.fori_loop(0, H // 16, body, 0, unroll=2)
            store_h[j] = pltpu.async_copy(
                ob[s], out_hbm.at[pl.ds(base + j * NCC, NCC)], st[s])
        for hh in store_h.values():
            hh.wait()

    return combine(ys, d0, d1, w0r, w1r)


@jax.jit
def kernel(x, gate_tensor, W_gate, W_up, W_down):
    gate_pad = jnp.zeros((H, LANES), jnp.float32).at[:, :E].set(gate_tensor)
    probs, cnt = _count_call(x, gate_pad)
    idx_out, w0b, w1b = _emit_call(probs, cnt)

    counts = cnt[0, :E].astype(jnp.int32)
    padded = ((counts + BR - 1) // BR) * BR
    ends = jnp.cumsum(padded)
    starts = jnp.arange(NB, dtype=jnp.int32) * BR
    be = jnp.sum((ends[None, :] <= starts[:, None]).astype(jnp.int32), axis=1)
    be = jnp.minimum(be, E - 1).astype(jnp.int32)

    d0 = idx_out[:, 0].reshape(NW, NCHD, NCD)
    d1 = idx_out[:, 1].reshape(NW, NCHD, NCD)
    d0c = idx_out[:, 0].reshape(NW, NCHC, NCC)
    d1c = idx_out[:, 1].reshape(NW, NCHC, NCC)
    w0r = w0b.reshape(NW, TPW, 16)
    w1r = w1b.reshape(NW, TPW, 16)

    xs = _dispatch_call(x, d0, d1)
    ys = _gmm_call(be, xs, W_gate, W_up, W_down)
    return _combine_call(ys, d0c, d1c, w0r, w1r)


# speedup vs baseline: 1.0328x; 1.0065x over previous
import functools

import jax
import jax.numpy as jnp
from jax import lax
from jax.experimental import pallas as pl
from jax.experimental.pallas import tpu as pltpu
from jax.experimental.pallas import tpu_sc as plsc

T = 2048
H = 2048
F = 512
E = 8
K = 2
LANES = 128
BT = 256
BR = 128
S_PAD = T * K + E * BR
NB = S_PAD // BR
NT = T // BT

_SC_CORES = 2
_SC_SUBCORES = 16
NW = _SC_CORES * _SC_SUBCORES
TPW = T // NW
NCD = 16
NCHD = TPW // NCD
NCC = 8
NCHC = TPW // NCC


def _count_kernel(x_ref, gate_ref, probs_ref, cnt_ref):
    t = pl.program_id(0)
    lane = lax.broadcasted_iota(jnp.int32, (BT, LANES), 1)
    valid = lane < E
    xv = x_ref[...]
    logits = jnp.dot(xv, gate_ref[...], preferred_element_type=jnp.float32)
    logits = jnp.where(valid, logits, -1e30)
    m1 = jnp.max(logits, axis=1, keepdims=True)
    p = jnp.where(valid, jnp.exp(logits - m1), 0.0)
    probs = p / jnp.sum(p, axis=1, keepdims=True)
    probs_ref[...] = probs

    v1 = jnp.max(probs, axis=1, keepdims=True)
    i1 = jnp.min(jnp.where(probs == v1, lane, LANES), axis=1, keepdims=True)
    probs2 = jnp.where(lane == i1, -1.0, probs)
    v2 = jnp.max(probs2, axis=1, keepdims=True)
    i2 = jnp.min(jnp.where(probs2 == v2, lane, LANES), axis=1, keepdims=True)
    onehot = ((lane == i1) | (lane == i2)).astype(jnp.float32)

    blk = jnp.sum(onehot, axis=0, keepdims=True)
    blk8 = jnp.broadcast_to(blk, (8, LANES))
    row = lax.broadcasted_iota(jnp.int32, (8, LANES), 0)
    blk8 = jnp.where(row == 0, blk8, 0.0)

    @pl.when(t == 0)
    def _init():
        cnt_ref[...] = blk8

    @pl.when(t != 0)
    def _acc():
        cnt_ref[...] += blk8


def _count_call(x, gate_pad):
    return pl.pallas_call(
        _count_kernel,
        grid=(NT,),
        in_specs=[
            pl.BlockSpec((BT, H), lambda t: (t, 0)),
            pl.BlockSpec((H, LANES), lambda t: (0, 0)),
        ],
        out_specs=[
            pl.BlockSpec((BT, LANES), lambda t: (t, 0)),
            pl.BlockSpec((8, LANES), lambda t: (0, 0)),
        ],
        out_shape=[
            jax.ShapeDtypeStruct((T, LANES), jnp.float32),
            jax.ShapeDtypeStruct((8, LANES), jnp.float32),
        ],
    )(x, gate_pad)


def _emit_kernel(probs_ref, cnt_ref, idx_ref, w0_ref, w1_ref, s_run):
    t = pl.program_id(0)
    lane = lax.broadcasted_iota(jnp.int32, (BT, LANES), 1)
    probs = probs_ref[...]
    v1 = jnp.max(probs, axis=1, keepdims=True)
    i1 = jnp.min(jnp.where(probs == v1, lane, LANES), axis=1, keepdims=True)
    probs2 = jnp.where(lane == i1, -1.0, probs)
    v2 = jnp.max(probs2, axis=1, keepdims=True)
    i2 = jnp.min(jnp.where(probs2 == v2, lane, LANES), axis=1, keepdims=True)
    onehot = ((lane == i1) | (lane == i2)).astype(jnp.float32)

    @pl.when(t == 0)
    def _offsets():
        counts = cnt_ref[0:1, :]
        padded = jnp.floor((counts + (BR - 1)) * (1.0 / BR)) * BR
        r = lax.broadcasted_iota(jnp.int32, (LANES, LANES), 0)
        c = lax.broadcasted_iota(jnp.int32, (LANES, LANES), 1)
        upper = (r < c).astype(jnp.float32)
        padded8 = jnp.broadcast_to(padded, (8, LANES))
        offs8 = jnp.dot(padded8, upper, preferred_element_type=jnp.float32)
        s_run[...] = offs8[0:1, :]

    base = s_run[...]
    rb = lax.broadcasted_iota(jnp.int32, (BT, BT), 0)
    cb = lax.broadcasted_iota(jnp.int32, (BT, BT), 1)
    lower = (cb < rb).astype(jnp.float32)
    rank = jnp.dot(lower, onehot, preferred_element_type=jnp.float32)
    pos = base + rank
    d0 = jnp.sum(jnp.where(lane == i1, pos, 0.0), axis=1, keepdims=True)
    d1 = jnp.sum(jnp.where(lane == i2, pos, 0.0), axis=1, keepdims=True)
    denom = v1 + v2 + 1e-20
    idx_ref[...] = (jnp.where(lane == 0, d0, 0.0)
                    + jnp.where(lane == 1, d1, 0.0)).astype(jnp.int32)
    w0_ref[...] = jnp.broadcast_to(v1 / denom, (BT, 16))
    w1_ref[...] = jnp.broadcast_to(v2 / denom, (BT, 16))
    s_run[...] += jnp.sum(onehot, axis=0, keepdims=True)


def _emit_call(probs, cnt):
    return pl.pallas_call(
        _emit_kernel,
        grid=(NT,),
        in_specs=[
            pl.BlockSpec((BT, LANES), lambda t: (t, 0)),
            pl.BlockSpec((8, LANES), lambda t: (0, 0)),
        ],
        out_specs=[
            pl.BlockSpec((BT, LANES), lambda t: (t, 0)),
            pl.BlockSpec((BT, 16), lambda t: (t, 0)),
            pl.BlockSpec((BT, 16), lambda t: (t, 0)),
        ],
        out_shape=[
            jax.ShapeDtypeStruct((T, LANES), jnp.int32),
            jax.ShapeDtypeStruct((T, 16), jnp.float32),
            jax.ShapeDtypeStruct((T, 16), jnp.float32),
        ],
        scratch_shapes=[pltpu.VMEM((1, LANES), jnp.float32)],
    )(probs, cnt)


def _dispatch_call(x, d0, d1):
    mesh = plsc.VectorSubcoreMesh(core_axis_name="c", subcore_axis_name="s")

    @functools.partial(
        pl.kernel,
        mesh=mesh,
        out_type=jax.ShapeDtypeStruct((S_PAD, H), jnp.float32),
        scratch_types=[
            pltpu.VMEM((NCHD, NCD), jnp.int32),
            pltpu.VMEM((NCHD, NCD), jnp.int32),
            pltpu.VMEM((NCD, H), jnp.float32),
            pltpu.VMEM((NCD, H), jnp.float32),
            pltpu.SemaphoreType.DMA,
            pltpu.SemaphoreType.DMA,
            pltpu.SemaphoreType.DMA,
            pltpu.SemaphoreType.DMA,
            pltpu.SemaphoreType.DMA,
            pltpu.SemaphoreType.DMA,
        ],
    )
    def dispatch(x_hbm, d0_hbm, d1_hbm, xs_hbm,
                 idx0_v, idx1_v, rows_a, rows_b,
                 ld_a, ld_b, s0_a, s0_b, s1_a, s1_b):
        wid = lax.axis_index("s") * _SC_CORES + lax.axis_index("c")
        base = wid * TPW
        pltpu.sync_copy(d0_hbm.at[wid], idx0_v)
        pltpu.sync_copy(d1_hbm.at[wid], idx1_v)
        rows = [rows_a, rows_b]
        ld = [ld_a, ld_b]
        s0 = [s0_a, s0_b]
        s1 = [s1_a, s1_b]

        def issue_load(j, s):
            return pltpu.async_copy(
                x_hbm.at[pl.ds(base + j * NCD, NCD)], rows[s], ld[s])

        load_h = {0: issue_load(0, 0)}
        scat_h = {}
        for j in range(NCHD):
            s = j & 1
            if j + 1 < NCHD:
                if j >= 1:
                    for hh in scat_h.pop(j - 1):
                        hh.wait()
                load_h[j + 1] = issue_load(j + 1, 1 - s)
            load_h.pop(j).wait()
            scat_h[j] = [
                pltpu.async_copy(rows[s], xs_hbm.at[idx0_v.at[j]], s0[s]),
                pltpu.async_copy(rows[s], xs_hbm.at[idx1_v.at[j]], s1[s]),
            ]
        for hs in scat_h.values():
            for hh in hs:
                hh.wait()

    return dispatch(x, d0, d1)


def _gmm_kernel(be_ref, xs_ref, wg_ref, wu_ref, wd_ref, ys_ref):
    del be_ref
    xv = xs_ref[...]
    g = jnp.dot(xv, wg_ref[0], preferred_element_type=jnp.float32)
    u = jnp.dot(xv, wu_ref[0], preferred_element_type=jnp.float32)
    h = g * (1.0 / (1.0 + jnp.exp(-g))) * u
    ys_ref[...] = jnp.dot(h, wd_ref[0], preferred_element_type=jnp.float32)


def _gmm_call(be, xs, W_gate, W_up, W_down):
    grid_spec = pltpu.PrefetchScalarGridSpec(
        num_scalar_prefetch=1,
        grid=(NB,),
        in_specs=[
            pl.BlockSpec((BR, H), lambda b, be: (b, 0)),
            pl.BlockSpec((1, H, F), lambda b, be: (be[b], 0, 0)),
            pl.BlockSpec((1, H, F), lambda b, be: (be[b], 0, 0)),
            pl.BlockSpec((1, F, H), lambda b, be: (be[b], 0, 0)),
        ],
        out_specs=pl.BlockSpec((BR, H), lambda b, be: (b, 0)),
    )
    return pl.pallas_call(
        _gmm_kernel,
        grid_spec=grid_spec,
        out_shape=jax.ShapeDtypeStruct((S_PAD, H), jnp.float32),
    )(be, xs, W_gate, W_up, W_down)


def _combine_call(ys, d0, d1, w0r, w1r):
    mesh = plsc.VectorSubcoreMesh(core_axis_name="c", subcore_axis_name="s")

    @functools.partial(
        pl.kernel,
        mesh=mesh,
        out_type=jax.ShapeDtypeStruct((T, H), jnp.float32),
        scratch_types=[
            pltpu.VMEM((NCHC, NCC), jnp.int32),
            pltpu.VMEM((NCHC, NCC), jnp.int32),
            pltpu.VMEM((TPW, 16), jnp.float32),
            pltpu.VMEM((TPW, 16), jnp.float32),
            pltpu.VMEM((NCC, H), jnp.float32),
            pltpu.VMEM((NCC, H), jnp.float32),
            pltpu.VMEM((NCC, H), jnp.float32),
            pltpu.VMEM((NCC, H), jnp.float32),
            pltpu.VMEM((NCC, H), jnp.float32),
            pltpu.VMEM((NCC, H), jnp.float32),
            pltpu.SemaphoreType.DMA,
            pltpu.SemaphoreType.DMA,
            pltpu.SemaphoreType.DMA,
            pltpu.SemaphoreType.DMA,
            pltpu.SemaphoreType.DMA,
            pltpu.SemaphoreType.DMA,
        ],
    )
    def combine(ys_hbm, d0_hbm, d1_hbm, w0_hbm, w1_hbm, out_hbm,
                idx0_v, idx1_v, w0_v, w1_v,
                b0_a, b0_b, b1_a, b1_b, ob_a, ob_b,
                g0_a, g0_b, g1_a, g1_b, st_a, st_b):
        wid = lax.axis_index("s") * _SC_CORES + lax.axis_index("c")
        base = wid * TPW
        pltpu.sync_copy(d0_hbm.at[wid], idx0_v)
        pltpu.sync_copy(d1_hbm.at[wid], idx1_v)
        pltpu.sync_copy(w0_hbm.at[wid], w0_v)
        pltpu.sync_copy(w1_hbm.at[wid], w1_v)
        b0 = [b0_a, b0_b]
        b1 = [b1_a, b1_b]
        ob = [ob_a, ob_b]
        g0 = [g0_a, g0_b]
        g1 = [g1_a, g1_b]
        st = [st_a, st_b]

        def issue_gather(j, s):
            return [
                pltpu.async_copy(ys_hbm.at[idx0_v.at[j]], b0[s], g0[s]),
                pltpu.async_copy(ys_hbm.at[idx1_v.at[j]], b1[s], g1[s]),
            ]

        gath_h = {0: issue_gather(0, 0)}
        store_h = {}
        for j in range(NCHC):
            s = j & 1
            if j + 1 < NCHC:
                gath_h[j + 1] = issue_gather(j + 1, 1 - s)
            for hh in gath_h.pop(j):
                hh.wait()
            if j >= 2:
                store_h.pop(j - 2).wait()
            for i in range(NCC):
                w0s = w0_v[j * NCC + i]
                w1s = w1_v[j * NCC + i]

                def body(q, _, s=s, i=i, w0s=w0s, w1s=w1s):
                    a = b0[s][i, pl.ds(q * 16, 16)]
                    b = b1[s][i, pl.ds(q * 16, 16)]
                    ob[s][i, pl.ds(q * 16, 16)] = w0s * a + w1s * b
                    return 0

                lax.fori_loop(0, H // 16, body, 0, unroll=2)
            store_h[j] = pltpu.async_copy(
                ob[s], out_hbm.at[pl.ds(base + j * NCC, NCC)], st[s])
        for hh in store_h.values():
            hh.wait()

    return combine(ys, d0, d1, w0r, w1r)


@jax.jit
def kernel(x, gate_tensor, W_gate, W_up, W_down):
    gate_pad = jnp.zeros((H, LANES), jnp.float32).at[:, :E].set(gate_tensor)
    probs, cnt = _count_call(x, gate_pad)
    idx_out, w0b, w1b = _emit_call(probs, cnt)

    counts = cnt[0, :E].astype(jnp.int32)
    padded = ((counts + BR - 1) // BR) * BR
    ends = jnp.cumsum(padded)
    starts = jnp.arange(NB, dtype=jnp.int32) * BR
    be = jnp.sum((ends[None, :] <= starts[:, None]).astype(jnp.int32), axis=1)
    be = jnp.minimum(be, E - 1).astype(jnp.int32)

    d0 = idx_out[:, 0].reshape(NW, NCHD, NCD)
    d1 = idx_out[:, 1].reshape(NW, NCHD, NCD)
    d0c = idx_out[:, 0].reshape(NW, NCHC, NCC)
    d1c = idx_out[:, 1].reshape(NW, NCHC, NCC)
    w0r = w0b.reshape(NW, TPW, 16)
    w1r = w1b.reshape(NW, TPW, 16)

    xs = _dispatch_call(x, d0, d1)
    ys = _gmm_call(be, xs, W_gate, W_up, W_down)
    return _combine_call(ys, d0c, d1c, w0r, w1r)
